# baseline (device time: 28710 ns/iter reference)
import jax
import jax.numpy as jnp
from jax import lax
from jax.experimental import pallas as pl
from jax.experimental.pallas import tpu as pltpu

N_DEV = 4
NC = 2


def kernel(t, W):
    m_per, k = t.shape
    _, n = W.shape
    H = m_per // 2
    Q = H // 2
    E = Q // 2
    KC = k // NC
    OC = n // NC

    def body(t_ref, w_ref, out_ref,
             red_a, red_b, recv_a1, recv_b1, recv_a2, recv_b2,
             send_sems, recv_sems):
        p = lax.axis_index("i")
        m1 = 3 - p
        m2 = p ^ 1
        k1 = p // 2
        k2 = p % 2
        j1 = (k1 + k2) % 2
        j2 = k1

        barrier_sem = pltpu.get_barrier_semaphore()
        for nbr in (m1, m2):
            pl.semaphore_signal(
                barrier_sem, inc=1,
                device_id=(nbr,), device_id_type=pl.DeviceIdType.MESH,
            )
        pl.semaphore_wait(barrier_sem, 2)

        started = []

        def xch(idx, src, dst, partner):
            rdma = pltpu.make_async_remote_copy(
                src_ref=src, dst_ref=dst,
                send_sem=send_sems.at[idx], recv_sem=recv_sems.at[idx],
                device_id=(partner,), device_id_type=pl.DeviceIdType.MESH,
            )
            rdma.start()
            started.append(rdma)
            return rdma

        def kc(c):
            return pl.ds(c * KC, KC)

        def oc(c):
            return pl.ds(c * OC, OC)

        row_a = k1 * Q + k2 * E
        row_b = H + j1 * Q + j2 * E
        send_ta = k1 * Q + (1 - k2) * E
        keep_ta = k1 * Q + k2 * E
        send_tb = H + j1 * Q + (1 - j2) * E
        keep_tb = H + j1 * Q + j2 * E

        r_a1 = [xch(0 + c, t_ref.at[pl.ds((1 - k1) * Q, Q), kc(c)],
                    recv_a1.at[:, kc(c)], m1) for c in range(NC)]
        r_b1 = [xch(2 + c, t_ref.at[pl.ds(H + (1 - j1) * Q, Q), kc(c)],
                    recv_b1.at[:, kc(c)], m2) for c in range(NC)]

        r_a2, r_b2 = [], []
        for c in range(NC):
            r_a1[c].wait_recv()
            red_a[:, kc(c)] = (t_ref[pl.ds(send_ta, E), kc(c)]
                               + recv_a1[pl.ds((1 - k2) * E, E), kc(c)])
            r_a2.append(xch(4 + c, red_a.at[:, kc(c)],
                            recv_a2.at[:, kc(c)], m2))
            r_b1[c].wait_recv()
            red_b[:, kc(c)] = (t_ref[pl.ds(send_tb, E), kc(c)]
                               + recv_b1[pl.ds((1 - j2) * E, E), kc(c)])
            r_b2.append(xch(6 + c, red_b.at[:, kc(c)],
                            recv_b2.at[:, kc(c)], m1))

        for r in r_a2:
            r.wait_recv()
        s_a = (t_ref[pl.ds(keep_ta, E), :]
               + recv_a1[pl.ds(k2 * E, E), :]
               + recv_a2[:, :])
        r_a3 = []
        for c in range(NC):
            out_ref[pl.ds(row_a, E), oc(c)] = jnp.dot(
                s_a, w_ref[:, oc(c)], preferred_element_type=jnp.float32)
            r_a3.append(xch(8 + c, out_ref.at[pl.ds(row_a, E), oc(c)],
                            out_ref.at[pl.ds(row_a, E), oc(c)], m2))
        for r in r_b2:
            r.wait_recv()
        s_b = (t_ref[pl.ds(keep_tb, E), :]
               + recv_b1[pl.ds(j2 * E, E), :]
               + recv_b2[:, :])
        r_b3 = []
        for c in range(NC):
            out_ref[pl.ds(row_b, E), oc(c)] = jnp.dot(
                s_b, w_ref[:, oc(c)], preferred_element_type=jnp.float32)
            r_b3.append(xch(10 + c, out_ref.at[pl.ds(row_b, E), oc(c)],
                            out_ref.at[pl.ds(row_b, E), oc(c)], m1))

        r_4 = []
        for c in range(NC):
            r_a3[c].wait_recv()
            r_4.append(xch(12 + c, out_ref.at[pl.ds(k1 * Q, Q), oc(c)],
                           out_ref.at[pl.ds(k1 * Q, Q), oc(c)], m1))
            r_b3[c].wait_recv()
            r_4.append(xch(14 + c, out_ref.at[pl.ds(H + j1 * Q, Q), oc(c)],
                           out_ref.at[pl.ds(H + j1 * Q, Q), oc(c)], m2))

        for r in r_4:
            r.wait_recv()
        for r in started:
            r.wait_send()

    return pl.pallas_call(
        body,
        out_shape=jax.ShapeDtypeStruct((m_per, n), jnp.float32),
        in_specs=[
            pl.BlockSpec(memory_space=pltpu.VMEM),
            pl.BlockSpec(memory_space=pltpu.VMEM),
        ],
        out_specs=pl.BlockSpec(memory_space=pltpu.VMEM),
        scratch_shapes=[
            pltpu.VMEM((E, k), jnp.float32),
            pltpu.VMEM((E, k), jnp.float32),
            pltpu.VMEM((Q, k), jnp.float32),
            pltpu.VMEM((Q, k), jnp.float32),
            pltpu.VMEM((E, k), jnp.float32),
            pltpu.VMEM((E, k), jnp.float32),
            pltpu.SemaphoreType.DMA((16,)),
            pltpu.SemaphoreType.DMA((16,)),
        ],
        compiler_params=pltpu.CompilerParams(collective_id=0),
    )(t, W)


# device time: 28513 ns/iter; 1.0069x vs baseline; 1.0069x over previous
import jax
import jax.numpy as jnp
from jax import lax
from jax.experimental import pallas as pl
from jax.experimental.pallas import tpu as pltpu

N_DEV = 4
NC = 2


def kernel(t, W):
    m_per, k = t.shape
    _, n = W.shape
    H = m_per // 2
    Q = H // 2
    E = Q // 2
    KC = k // NC
    OC = n // NC

    def body(t_ref, w_ref, out_ref,
             red_a, red_b, recv_a1, recv_b1, recv_a2, recv_b2,
             send_sems, recv_sems):
        p = lax.axis_index("i")
        m1 = 3 - p
        m2 = p ^ 1
        k1 = p // 2
        k2 = p % 2
        j1 = (k1 + k2) % 2
        j2 = k1

        barrier_sem = pltpu.get_barrier_semaphore()
        for nbr in (m1, m2):
            pl.semaphore_signal(
                barrier_sem, inc=1,
                device_id=(nbr,), device_id_type=pl.DeviceIdType.MESH,
            )
        pl.semaphore_wait(barrier_sem, 2)

        started = []

        def xch(idx, src, dst, partner):
            rdma = pltpu.make_async_remote_copy(
                src_ref=src, dst_ref=dst,
                send_sem=send_sems.at[idx], recv_sem=recv_sems.at[idx],
                device_id=(partner,), device_id_type=pl.DeviceIdType.MESH,
            )
            rdma.start()
            started.append(rdma)
            return rdma

        def kc(c):
            return pl.ds(c * KC, KC)

        def oc(c):
            return pl.ds(c * OC, OC)

        row_a = k1 * Q + k2 * E
        row_b = H + j1 * Q + j2 * E
        send_ta = k1 * Q + (1 - k2) * E
        keep_ta = k1 * Q + k2 * E
        send_tb = H + j1 * Q + (1 - j2) * E
        keep_tb = H + j1 * Q + j2 * E

        r_a1 = [xch(0 + c, t_ref.at[pl.ds((1 - k1) * Q, Q), kc(c)],
                    recv_a1.at[:, kc(c)], m1) for c in range(NC)]
        r_b1 = [xch(2 + c, t_ref.at[pl.ds(H + (1 - j1) * Q, Q), kc(c)],
                    recv_b1.at[:, kc(c)], m2) for c in range(NC)]

        r_a2, r_b2 = [], []
        for c in range(NC):
            r_a1[c].wait_recv()
            red_a[:, kc(c)] = (t_ref[pl.ds(send_ta, E), kc(c)]
                               + recv_a1[pl.ds((1 - k2) * E, E), kc(c)])
            r_a2.append(xch(4 + c, red_a.at[:, kc(c)],
                            recv_a2.at[:, kc(c)], m2))
            r_b1[c].wait_recv()
            red_b[:, kc(c)] = (t_ref[pl.ds(send_tb, E), kc(c)]
                               + recv_b1[pl.ds((1 - j2) * E, E), kc(c)])
            r_b2.append(xch(6 + c, red_b.at[:, kc(c)],
                            recv_b2.at[:, kc(c)], m1))

        for r in r_a2:
            r.wait_recv()
        s_a = (t_ref[pl.ds(keep_ta, E), :]
               + recv_a1[pl.ds(k2 * E, E), :]
               + recv_a2[:, :])
        for r in r_b2:
            r.wait_recv()
        s_b = (t_ref[pl.ds(keep_tb, E), :]
               + recv_b1[pl.ds(j2 * E, E), :]
               + recv_b2[:, :])
        r_a3, r_b3 = [], []
        for c in range(NC):
            out_ref[pl.ds(row_a, E), oc(c)] = jnp.dot(
                s_a, w_ref[:, oc(c)], preferred_element_type=jnp.float32)
            r_a3.append(xch(8 + c, out_ref.at[pl.ds(row_a, E), oc(c)],
                            out_ref.at[pl.ds(row_a, E), oc(c)], m2))
            out_ref[pl.ds(row_b, E), oc(c)] = jnp.dot(
                s_b, w_ref[:, oc(c)], preferred_element_type=jnp.float32)
            r_b3.append(xch(10 + c, out_ref.at[pl.ds(row_b, E), oc(c)],
                            out_ref.at[pl.ds(row_b, E), oc(c)], m1))

        r_4 = []
        for c in range(NC):
            r_a3[c].wait_recv()
            r_4.append(xch(12 + c, out_ref.at[pl.ds(k1 * Q, Q), oc(c)],
                           out_ref.at[pl.ds(k1 * Q, Q), oc(c)], m1))
            r_b3[c].wait_recv()
            r_4.append(xch(14 + c, out_ref.at[pl.ds(H + j1 * Q, Q), oc(c)],
                           out_ref.at[pl.ds(H + j1 * Q, Q), oc(c)], m2))

        for r in r_4:
            r.wait_recv()
        for r in started:
            r.wait_send()

    return pl.pallas_call(
        body,
        out_shape=jax.ShapeDtypeStruct((m_per, n), jnp.float32),
        in_specs=[
            pl.BlockSpec(memory_space=pltpu.VMEM),
            pl.BlockSpec(memory_space=pltpu.VMEM),
        ],
        out_specs=pl.BlockSpec(memory_space=pltpu.VMEM),
        scratch_shapes=[
            pltpu.VMEM((E, k), jnp.float32),
            pltpu.VMEM((E, k), jnp.float32),
            pltpu.VMEM((Q, k), jnp.float32),
            pltpu.VMEM((Q, k), jnp.float32),
            pltpu.VMEM((E, k), jnp.float32),
            pltpu.VMEM((E, k), jnp.float32),
            pltpu.SemaphoreType.DMA((16,)),
            pltpu.SemaphoreType.DMA((16,)),
        ],
        compiler_params=pltpu.CompilerParams(collective_id=0),
    )(t, W)
